# initial kernel scaffold (unmeasured)
import jax
import jax.numpy as jnp
from jax import lax
from jax.experimental import pallas as pl
from jax.experimental.pallas import tpu as pltpu

N_DEV = 16


def _ag_body(x_ref, w_ref, xg_ref, wg_ref, xc, wc, send_sems, recv_sems):
    my = lax.axis_index("i")
    left = lax.rem(my + N_DEV - 1, N_DEV)
    right = lax.rem(my + 1, N_DEV)
    m, k_per = x_ref.shape
    n = w_ref.shape[1]

    barrier_sem = pltpu.get_barrier_semaphore()
    for nbr in (left, right):
        pl.semaphore_signal(
            barrier_sem, inc=1,
            device_id=(nbr,), device_id_type=pl.DeviceIdType.MESH,
        )
    pl.semaphore_wait(barrier_sem, 2)

    xg_ref[:, pl.ds(my * k_per, k_per)] = x_ref[...]
    wg_ref[pl.ds(my * k_per, k_per), :] = w_ref[...]
    xc[0] = x_ref[...]
    wc[0] = w_ref[...]

    for h in range(N_DEV - 1):
        s = h % 2
        r = (h + 1) % 2
        rx = pltpu.make_async_remote_copy(
            src_ref=xc.at[s], dst_ref=xc.at[r],
            send_sem=send_sems.at[0, s], recv_sem=recv_sems.at[0, r],
            device_id=(right,), device_id_type=pl.DeviceIdType.MESH,
        )
        rw = pltpu.make_async_remote_copy(
            src_ref=wc.at[s], dst_ref=wc.at[r],
            send_sem=send_sems.at[1, s], recv_sem=recv_sems.at[1, r],
            device_id=(right,), device_id_type=pl.DeviceIdType.MESH,
        )
        rx.start()
        rw.start()
        rx.wait()
        rw.wait()
        origin = lax.rem(my + N_DEV - 1 - h, N_DEV)
        xg_ref[:, pl.ds(origin * k_per, k_per)] = xc[r]
        wg_ref[pl.ds(origin * k_per, k_per), :] = wc[r]


def kernel(x, w_mat, scale_x, scale_w):
    m, k_per = x.shape
    n = w_mat.shape[1]
    k = k_per * N_DEV

    xg, wg = pl.pallas_call(
        _ag_body,
        out_shape=(
            jax.ShapeDtypeStruct((m, k), x.dtype),
            jax.ShapeDtypeStruct((k, n), w_mat.dtype),
        ),
        in_specs=[
            pl.BlockSpec(memory_space=pltpu.VMEM),
            pl.BlockSpec(memory_space=pltpu.VMEM),
        ],
        out_specs=(
            pl.BlockSpec(memory_space=pltpu.VMEM),
            pl.BlockSpec(memory_space=pltpu.VMEM),
        ),
        scratch_shapes=[
            pltpu.VMEM((2, m, k_per), x.dtype),
            pltpu.VMEM((2, k_per, n), w_mat.dtype),
            pltpu.SemaphoreType.DMA((2, 2)),
            pltpu.SemaphoreType.DMA((2, 2)),
        ],
        compiler_params=pltpu.CompilerParams(collective_id=0),
    )(x, w_mat)

    acc = jnp.dot(xg, wg, preferred_element_type=jnp.float32)
    return acc * (scale_x[0] * scale_w[0])


# baseline (device time: 747298 ns/iter reference)
import jax
import jax.numpy as jnp
from jax import lax
from jax.experimental import pallas as pl
from jax.experimental.pallas import tpu as pltpu

N_DEV = 16


def _ag_body(x_ref, w_ref, xg_ref, wg_ref, send_sems, recv_sems):
    my = lax.axis_index("i")
    left = lax.rem(my + N_DEV - 1, N_DEV)
    right = lax.rem(my + 1, N_DEV)
    m, k_per = x_ref.shape
    n = w_ref.shape[1]

    barrier_sem = pltpu.get_barrier_semaphore()
    for nbr in (left, right):
        pl.semaphore_signal(
            barrier_sem, inc=1,
            device_id=(nbr,), device_id_type=pl.DeviceIdType.MESH,
        )
    pl.semaphore_wait(barrier_sem, 2)

    my_off = pl.multiple_of(my * k_per, k_per)
    xg_ref[:, pl.ds(my_off, k_per)] = x_ref[...]
    wg_ref[pl.ds(my_off, k_per), :] = w_ref[...]

    for h in range(N_DEV - 1):
        origin = lax.rem(my + N_DEV - h, N_DEV)
        o_off = pl.multiple_of(origin * k_per, k_per)
        rx = pltpu.make_async_remote_copy(
            src_ref=xg_ref.at[:, pl.ds(o_off, k_per)],
            dst_ref=xg_ref.at[:, pl.ds(o_off, k_per)],
            send_sem=send_sems.at[0, h], recv_sem=recv_sems.at[0, h],
            device_id=(right,), device_id_type=pl.DeviceIdType.MESH,
        )
        rw = pltpu.make_async_remote_copy(
            src_ref=wg_ref.at[pl.ds(o_off, k_per), :],
            dst_ref=wg_ref.at[pl.ds(o_off, k_per), :],
            send_sem=send_sems.at[1, h], recv_sem=recv_sems.at[1, h],
            device_id=(right,), device_id_type=pl.DeviceIdType.MESH,
        )
        rx.start()
        rw.start()
        rx.wait()
        rw.wait()


def _mm_body(x_ref, w_ref, s_ref, o_ref):
    o_ref[...] = (
        jnp.dot(x_ref[...], w_ref[...], preferred_element_type=jnp.float32)
        * s_ref[0, 0]
    )


def kernel(x, w_mat, scale_x, scale_w):
    x = x.astype(jnp.float8_e5m2)
    w_mat = w_mat.astype(jnp.float8_e5m2)
    m, k_per = x.shape
    n = w_mat.shape[1]
    k = k_per * N_DEV

    xg, wg = pl.pallas_call(
        _ag_body,
        out_shape=(
            jax.ShapeDtypeStruct((m, k), x.dtype),
            jax.ShapeDtypeStruct((k, n), w_mat.dtype),
        ),
        in_specs=[
            pl.BlockSpec(memory_space=pltpu.VMEM),
            pl.BlockSpec(memory_space=pltpu.VMEM),
        ],
        out_specs=(
            pl.BlockSpec(memory_space=pltpu.VMEM),
            pl.BlockSpec(memory_space=pltpu.VMEM),
        ),
        scratch_shapes=[
            pltpu.SemaphoreType.DMA((2, N_DEV - 1)),
            pltpu.SemaphoreType.DMA((2, N_DEV - 1)),
        ],
        compiler_params=pltpu.CompilerParams(collective_id=0),
    )(x, w_mat)

    s = (scale_x * scale_w).reshape(1, 1)
    bm, bn = 512, 1024
    out = pl.pallas_call(
        _mm_body,
        grid=(m // bm, n // bn),
        in_specs=[
            pl.BlockSpec((bm, k), lambda i, j: (i, 0)),
            pl.BlockSpec((k, bn), lambda i, j: (0, j)),
            pl.BlockSpec(memory_space=pltpu.SMEM),
        ],
        out_specs=pl.BlockSpec((bm, bn), lambda i, j: (i, j)),
        out_shape=jax.ShapeDtypeStruct((m, n), jnp.float32),
    )(xg, wg, s)
    return out


# device time: 498380 ns/iter; 1.4995x vs baseline; 1.4995x over previous
import jax
import jax.numpy as jnp
from jax import lax
from jax.experimental import pallas as pl
from jax.experimental.pallas import tpu as pltpu

N_DEV = 16

_RING = [0, 1, 2, 3, 7, 6, 5, 9, 10, 11, 15, 14, 13, 12, 8, 4]
_RING_INV = [0] * N_DEV
for _p, _d in enumerate(_RING):
    _RING_INV[_d] = _p

N_FWD = N_DEV // 2
N_BWD = N_DEV - 1 - N_FWD


def _lut(idx, table):
    v = jnp.int32(table[0])
    for t in range(1, len(table)):
        v = jnp.where(idx == t, jnp.int32(table[t]), v)
    return v


def _ag_body(x_ref, w_ref, xg_ref, wg_ref, send_sems, recv_sems):
    my = lax.axis_index("i")
    m, k_per = x_ref.shape
    n = w_ref.shape[1]

    pos = _lut(my, _RING_INV)
    right = _lut(lax.rem(pos + 1, N_DEV), _RING)
    left = _lut(lax.rem(pos + N_DEV - 1, N_DEV), _RING)

    barrier_sem = pltpu.get_barrier_semaphore()
    for nbr in (left, right):
        pl.semaphore_signal(
            barrier_sem, inc=1,
            device_id=(nbr,), device_id_type=pl.DeviceIdType.MESH,
        )
    pl.semaphore_wait(barrier_sem, 2)

    my_off = pl.multiple_of(my * k_per, k_per)
    xg_ref[:, pl.ds(my_off, k_per)] = x_ref[...]
    wg_ref[pl.ds(my_off, k_per), :] = w_ref[...]

    for h in range(N_FWD):
        rdmas = []
        for d in range(2):
            if d == 0:
                origin = _lut(lax.rem(pos + N_DEV - h, N_DEV), _RING)
                tgt = right
            else:
                if h >= N_BWD:
                    continue
                origin = _lut(lax.rem(pos + h, N_DEV), _RING)
                tgt = left
            o_off = pl.multiple_of(origin * k_per, k_per)
            rdmas.append(pltpu.make_async_remote_copy(
                src_ref=xg_ref.at[:, pl.ds(o_off, k_per)],
                dst_ref=xg_ref.at[:, pl.ds(o_off, k_per)],
                send_sem=send_sems.at[0, d, h], recv_sem=recv_sems.at[0, d, h],
                device_id=(tgt,), device_id_type=pl.DeviceIdType.MESH,
            ))
            rdmas.append(pltpu.make_async_remote_copy(
                src_ref=wg_ref.at[pl.ds(o_off, k_per), :],
                dst_ref=wg_ref.at[pl.ds(o_off, k_per), :],
                send_sem=send_sems.at[1, d, h], recv_sem=recv_sems.at[1, d, h],
                device_id=(tgt,), device_id_type=pl.DeviceIdType.MESH,
            ))
        for r in rdmas:
            r.start()
        for r in rdmas:
            r.wait()


def _mm_body(x_ref, w_ref, s_ref, o_ref):
    o_ref[...] = (
        jnp.dot(x_ref[...], w_ref[...], preferred_element_type=jnp.float32)
        * s_ref[0, 0]
    )


def kernel(x, w_mat, scale_x, scale_w):
    x = x.astype(jnp.float8_e5m2)
    w_mat = w_mat.astype(jnp.float8_e5m2)
    m, k_per = x.shape
    n = w_mat.shape[1]
    k = k_per * N_DEV

    xg, wg = pl.pallas_call(
        _ag_body,
        out_shape=(
            jax.ShapeDtypeStruct((m, k), x.dtype),
            jax.ShapeDtypeStruct((k, n), w_mat.dtype),
        ),
        in_specs=[
            pl.BlockSpec(memory_space=pltpu.VMEM),
            pl.BlockSpec(memory_space=pltpu.VMEM),
        ],
        out_specs=(
            pl.BlockSpec(memory_space=pltpu.VMEM),
            pl.BlockSpec(memory_space=pltpu.VMEM),
        ),
        scratch_shapes=[
            pltpu.SemaphoreType.DMA((2, 2, N_FWD)),
            pltpu.SemaphoreType.DMA((2, 2, N_FWD)),
        ],
        compiler_params=pltpu.CompilerParams(collective_id=0),
    )(x, w_mat)

    s = (scale_x * scale_w).reshape(1, 1)
    bm, bn = 512, 1024
    out = pl.pallas_call(
        _mm_body,
        grid=(m // bm, n // bn),
        in_specs=[
            pl.BlockSpec((bm, k), lambda i, j: (i, 0)),
            pl.BlockSpec((k, bn), lambda i, j: (0, j)),
            pl.BlockSpec(memory_space=pltpu.SMEM),
        ],
        out_specs=pl.BlockSpec((bm, bn), lambda i, j: (i, j)),
        out_shape=jax.ShapeDtypeStruct((m, n), jnp.float32),
    )(xg, wg, s)
    return out


# device time: 478021 ns/iter; 1.5633x vs baseline; 1.0426x over previous
import jax
import jax.numpy as jnp
from jax import lax
from jax.experimental import pallas as pl
from jax.experimental.pallas import tpu as pltpu

N_DEV = 16

_RING = [0, 1, 2, 3, 7, 6, 5, 9, 10, 11, 15, 14, 13, 12, 8, 4]
_RING_INV = [0] * N_DEV
for _p, _d in enumerate(_RING):
    _RING_INV[_d] = _p

N_FWD = N_DEV // 2
N_BWD = N_DEV - 1 - N_FWD


def _lut(idx, table):
    v = jnp.int32(table[0])
    for t in range(1, len(table)):
        v = jnp.where(idx == t, jnp.int32(table[t]), v)
    return v


def _ag_body(x_ref, w_ref, xg_ref, wg_ref, send_sems, recv_sems):
    my = lax.axis_index("i")
    m, k_per = x_ref.shape
    n = w_ref.shape[1]

    pos = _lut(my, _RING_INV)
    right = _lut(lax.rem(pos + 1, N_DEV), _RING)
    left = _lut(lax.rem(pos + N_DEV - 1, N_DEV), _RING)

    barrier_sem = pltpu.get_barrier_semaphore()
    for nbr in (left, right):
        pl.semaphore_signal(
            barrier_sem, inc=1,
            device_id=(nbr,), device_id_type=pl.DeviceIdType.MESH,
        )
    pl.semaphore_wait(barrier_sem, 2)

    my_off = pl.multiple_of(my * k_per, k_per)
    xg_ref[:, pl.ds(my_off, k_per)] = x_ref[...]
    wg_ref[pl.ds(my_off, k_per), :] = w_ref[...]

    for h in range(N_FWD):
        rdmas = []
        for d in range(2):
            if d == 0:
                origin = _lut(lax.rem(pos + N_DEV - h, N_DEV), _RING)
                tgt = right
            else:
                if h >= N_BWD:
                    continue
                origin = _lut(lax.rem(pos + h, N_DEV), _RING)
                tgt = left
            o_off = pl.multiple_of(origin * k_per, k_per)
            rdmas.append(pltpu.make_async_remote_copy(
                src_ref=xg_ref.at[:, pl.ds(o_off, k_per)],
                dst_ref=xg_ref.at[:, pl.ds(o_off, k_per)],
                send_sem=send_sems.at[0, d, h], recv_sem=recv_sems.at[0, d, h],
                device_id=(tgt,), device_id_type=pl.DeviceIdType.MESH,
            ))
            rdmas.append(pltpu.make_async_remote_copy(
                src_ref=wg_ref.at[pl.ds(o_off, k_per), :],
                dst_ref=wg_ref.at[pl.ds(o_off, k_per), :],
                send_sem=send_sems.at[1, d, h], recv_sem=recv_sems.at[1, d, h],
                device_id=(tgt,), device_id_type=pl.DeviceIdType.MESH,
            ))
        for r in rdmas:
            r.start()
        for r in rdmas:
            r.wait()


def _mm_body(x_ref, w_ref, s_ref, o_ref):
    o_ref[...] = (
        jnp.dot(x_ref[...], w_ref[...], preferred_element_type=jnp.float32)
        * s_ref[0, 0]
    )


def kernel(x, w_mat, scale_x, scale_w):
    x = x.astype(jnp.float8_e5m2)
    w_mat = w_mat.astype(jnp.float8_e5m2)
    m, k_per = x.shape
    n = w_mat.shape[1]
    k = k_per * N_DEV

    xg, wg = pl.pallas_call(
        _ag_body,
        out_shape=(
            jax.ShapeDtypeStruct((m, k), x.dtype),
            jax.ShapeDtypeStruct((k, n), w_mat.dtype),
        ),
        in_specs=[
            pl.BlockSpec(memory_space=pltpu.VMEM),
            pl.BlockSpec(memory_space=pltpu.VMEM),
        ],
        out_specs=(
            pl.BlockSpec(memory_space=pltpu.VMEM),
            pl.BlockSpec(memory_space=pltpu.VMEM),
        ),
        scratch_shapes=[
            pltpu.SemaphoreType.DMA((2, 2, N_FWD)),
            pltpu.SemaphoreType.DMA((2, 2, N_FWD)),
        ],
        compiler_params=pltpu.CompilerParams(collective_id=0),
    )(x, w_mat)

    s = (scale_x * scale_w).reshape(1, 1)
    bn = 512
    out = pl.pallas_call(
        _mm_body,
        grid=(n // bn,),
        in_specs=[
            pl.BlockSpec((m, k), lambda j: (0, 0)),
            pl.BlockSpec((k, bn), lambda j: (0, j)),
            pl.BlockSpec(memory_space=pltpu.SMEM),
        ],
        out_specs=pl.BlockSpec((m, bn), lambda j: (0, j)),
        out_shape=jax.ShapeDtypeStruct((m, n), jnp.float32),
    )(xg, wg, s)
    return out


# device time: 464765 ns/iter; 1.6079x vs baseline; 1.0285x over previous
import jax
import jax.numpy as jnp
from jax import lax
from jax.experimental import pallas as pl
from jax.experimental.pallas import tpu as pltpu

N_DEV = 16

_RING = [0, 1, 2, 3, 7, 6, 5, 9, 10, 11, 15, 14, 13, 12, 8, 4]
_RING_INV = [0] * N_DEV
for _p, _d in enumerate(_RING):
    _RING_INV[_d] = _p

N_FWD = N_DEV // 2
N_BWD = N_DEV - 1 - N_FWD


def _lut(idx, table):
    v = jnp.int32(table[0])
    for t in range(1, len(table)):
        v = jnp.where(idx == t, jnp.int32(table[t]), v)
    return v


def _ag_body(x_ref, w_ref, xg_ref, wg_ref, send_sems, recv_sems):
    my = lax.axis_index("i")
    m, k_per = x_ref.shape
    n = w_ref.shape[1]

    pos = _lut(my, _RING_INV)
    right = _lut(lax.rem(pos + 1, N_DEV), _RING)
    left = _lut(lax.rem(pos + N_DEV - 1, N_DEV), _RING)

    barrier_sem = pltpu.get_barrier_semaphore()
    for nbr in (left, right):
        pl.semaphore_signal(
            barrier_sem, inc=1,
            device_id=(nbr,), device_id_type=pl.DeviceIdType.MESH,
        )
    pl.semaphore_wait(barrier_sem, 2)

    my_off = pl.multiple_of(my * k_per, k_per)
    xg_ref[:, pl.ds(my_off, k_per)] = x_ref[...]
    wg_ref[pl.ds(my_off, k_per), :] = w_ref[...]

    def _hop_rdmas(h):
        rdmas = []
        for d in range(2):
            if d == 0:
                if h >= N_FWD:
                    continue
                origin = _lut(lax.rem(pos + N_DEV - h, N_DEV), _RING)
                tgt = right
            else:
                if h >= N_BWD:
                    continue
                origin = _lut(lax.rem(pos + h, N_DEV), _RING)
                tgt = left
            o_off = pl.multiple_of(origin * k_per, k_per)
            rdmas.append(pltpu.make_async_remote_copy(
                src_ref=xg_ref.at[:, pl.ds(o_off, k_per)],
                dst_ref=xg_ref.at[:, pl.ds(o_off, k_per)],
                send_sem=send_sems.at[0, d, h], recv_sem=recv_sems.at[0, d, h],
                device_id=(tgt,), device_id_type=pl.DeviceIdType.MESH,
            ))
            rdmas.append(pltpu.make_async_remote_copy(
                src_ref=wg_ref.at[pl.ds(o_off, k_per), :],
                dst_ref=wg_ref.at[pl.ds(o_off, k_per), :],
                send_sem=send_sems.at[1, d, h], recv_sem=recv_sems.at[1, d, h],
                device_id=(tgt,), device_id_type=pl.DeviceIdType.MESH,
            ))
        return rdmas

    prev = _hop_rdmas(0)
    for r in prev:
        r.start()
    for h in range(1, N_FWD):
        cur = _hop_rdmas(h)
        for rp, rc in zip(prev, cur):
            rp.wait()
            rc.start()
        for rp in prev[len(cur):]:
            rp.wait()
        prev = cur
    for rp in prev:
        rp.wait()


def _mm_body(x_ref, w_ref, s_ref, o_ref):
    o_ref[...] = (
        jnp.dot(x_ref[...], w_ref[...], preferred_element_type=jnp.float32)
        * s_ref[0, 0]
    )


def kernel(x, w_mat, scale_x, scale_w):
    x = x.astype(jnp.float8_e5m2)
    w_mat = w_mat.astype(jnp.float8_e5m2)
    m, k_per = x.shape
    n = w_mat.shape[1]
    k = k_per * N_DEV

    xg, wg = pl.pallas_call(
        _ag_body,
        out_shape=(
            jax.ShapeDtypeStruct((m, k), x.dtype),
            jax.ShapeDtypeStruct((k, n), w_mat.dtype),
        ),
        in_specs=[
            pl.BlockSpec(memory_space=pltpu.VMEM),
            pl.BlockSpec(memory_space=pltpu.VMEM),
        ],
        out_specs=(
            pl.BlockSpec(memory_space=pltpu.VMEM),
            pl.BlockSpec(memory_space=pltpu.VMEM),
        ),
        scratch_shapes=[
            pltpu.SemaphoreType.DMA((2, 2, N_FWD)),
            pltpu.SemaphoreType.DMA((2, 2, N_FWD)),
        ],
        compiler_params=pltpu.CompilerParams(collective_id=0),
    )(x, w_mat)

    s = (scale_x * scale_w).reshape(1, 1)
    bn = 512
    out = pl.pallas_call(
        _mm_body,
        grid=(n // bn,),
        in_specs=[
            pl.BlockSpec((m, k), lambda j: (0, 0)),
            pl.BlockSpec((k, bn), lambda j: (0, j)),
            pl.BlockSpec(memory_space=pltpu.SMEM),
        ],
        out_specs=pl.BlockSpec((m, bn), lambda j: (0, j)),
        out_shape=jax.ShapeDtypeStruct((m, n), jnp.float32),
    )(xg, wg, s)
    return out
